# Initial kernel scaffold; baseline (speedup 1.0000x reference)
import jax
import jax.numpy as jnp
from jax.experimental import pallas as pl


def kernel(indices, weight):
    raise NotImplementedError("write your pallas kernel here")


# speedup vs baseline: 4.1882x; 4.1882x over previous
import functools

import jax
import jax.numpy as jnp
from jax import lax
from jax.experimental import pallas as pl
from jax.experimental.pallas import tpu as pltpu
from jax.experimental.pallas import tpu_sc as plsc

NW = 32
C = 128
K = 2


def _lookup_call(tot, d, nch, ng, per_w):
    mesh = plsc.VectorSubcoreMesh(core_axis_name="c", subcore_axis_name="s")

    @functools.partial(
        pl.kernel,
        out_type=jax.ShapeDtypeStruct((tot, d), jnp.float32),
        mesh=mesh,
        scratch_types=[
            pltpu.VMEM((nch, C), jnp.int32),
            pltpu.VMEM((C, d), jnp.float32),
            pltpu.VMEM((C, d), jnp.float32),
            pltpu.VMEM((C, d), jnp.float32),
            pltpu.VMEM((C, d), jnp.float32),
            pltpu.SemaphoreType.DMA,
            pltpu.SemaphoreType.DMA,
            pltpu.SemaphoreType.DMA,
            pltpu.SemaphoreType.DMA,
        ],
    )
    def k(idx_hbm, table_hbm, out_hbm, idx_all, b00, b01, b10, b11,
          gs0, gs1, ws0, ws1):
        bufs = ((b00, b01), (b10, b11))
        gsem = (gs0, gs1)
        wsem = (ws0, ws1)
        wid = lax.axis_index("s") * 2 + lax.axis_index("c")
        base = wid * per_w

        pltpu.sync_copy(idx_hbm.at[wid], idx_all)

        def gather(gr, h):
            return [pltpu.make_async_copy(
                table_hbm.at[idx_all.at[gr * K + b]], bufs[h][b], gsem[h])
                for b in range(K)]

        def write(gr, h):
            return [pltpu.make_async_copy(
                bufs[h][b], out_hbm.at[pl.ds(base + (gr * K + b) * C, C)],
                wsem[h]) for b in range(K)]

        for cp in gather(0, 0):
            cp.start()

        @pl.loop(0, ng, step=2)
        def _(g):
            for h in range(2):
                gr = g + h
                for cp in gather(gr, h):
                    cp.wait()
                for cp in write(gr, h):
                    cp.start()

                @pl.when(gr >= 1)
                def _():
                    for cp in write(gr - 1, 1 - h):
                        cp.wait()

                @pl.when(gr + 1 < ng)
                def _():
                    for cp in gather(gr + 1, 1 - h):
                        cp.start()

        for cp in write(ng - 1, 1):
            cp.wait()

    return k


def kernel(indices, weight):
    b, hist = indices.shape
    v, d = weight.shape
    tot = b * hist
    per_w = tot // NW
    nch = per_w // C
    ng = nch // K
    idx3 = indices.reshape(NW, nch, C)
    out = _lookup_call(tot, d, nch, ng, per_w)(idx3, weight)
    return out.reshape(b, hist, d)


# speedup vs baseline: 15.4682x; 3.6932x over previous
import functools

import jax
import jax.numpy as jnp
from jax import lax
from jax.experimental import pallas as pl
from jax.experimental.pallas import tpu as pltpu
from jax.experimental.pallas import tpu_sc as plsc

NW = 32
C = 128
K = 2


def _lookup_call(tot, v, d, nch, ng, per_w):
    mesh = plsc.VectorSubcoreMesh(core_axis_name="c", subcore_axis_name="s")

    @functools.partial(
        pl.kernel,
        out_type=jax.ShapeDtypeStruct((tot, d), jnp.float32),
        mesh=mesh,
        scratch_types=[
            pltpu.VMEM_SHARED((v, d), jnp.float32),
            pltpu.VMEM((nch, C), jnp.int32),
            pltpu.VMEM((C, d), jnp.float32),
            pltpu.VMEM((C, d), jnp.float32),
            pltpu.VMEM((C, d), jnp.float32),
            pltpu.VMEM((C, d), jnp.float32),
            pltpu.SemaphoreType.DMA,
            pltpu.SemaphoreType.DMA,
            pltpu.SemaphoreType.DMA,
            pltpu.SemaphoreType.DMA,
        ],
    )
    def k(idx_hbm, table_hbm, out_hbm, table_sp, idx_all, b00, b01, b10, b11,
          gs0, gs1, ws0, ws1):
        bufs = ((b00, b01), (b10, b11))
        gsem = (gs0, gs1)
        wsem = (ws0, ws1)
        sid = lax.axis_index("s")
        wid = sid * 2 + lax.axis_index("c")
        base = wid * per_w

        @pl.when(sid == 0)
        def _():
            pltpu.sync_copy(table_hbm, table_sp)

        pltpu.sync_copy(idx_hbm.at[wid], idx_all)
        plsc.subcore_barrier()

        def gather(gr, h):
            return [pltpu.make_async_copy(
                table_sp.at[idx_all.at[gr * K + b]], bufs[h][b], gsem[h])
                for b in range(K)]

        def write(gr, h):
            return [pltpu.make_async_copy(
                bufs[h][b], out_hbm.at[pl.ds(base + (gr * K + b) * C, C)],
                wsem[h]) for b in range(K)]

        for cp in gather(0, 0):
            cp.start()

        @pl.loop(0, ng, step=2)
        def _(g):
            for h in range(2):
                gr = g + h
                for cp in gather(gr, h):
                    cp.wait()
                for cp in write(gr, h):
                    cp.start()

                @pl.when(gr >= 1)
                def _():
                    for cp in write(gr - 1, 1 - h):
                        cp.wait()

                @pl.when(gr + 1 < ng)
                def _():
                    for cp in gather(gr + 1, 1 - h):
                        cp.start()

        for cp in write(ng - 1, 1):
            cp.wait()

    return k


def kernel(indices, weight):
    b, hist = indices.shape
    v, d = weight.shape
    tot = b * hist
    per_w = tot // NW
    nch = per_w // C
    ng = nch // K
    idx3 = indices.reshape(NW, nch, C)
    out = _lookup_call(tot, v, d, nch, ng, per_w)(idx3, weight)
    return out.reshape(b, hist, d)


# speedup vs baseline: 15.5222x; 1.0035x over previous
import functools

import jax
import jax.numpy as jnp
from jax import lax
from jax.experimental import pallas as pl
from jax.experimental.pallas import tpu as pltpu
from jax.experimental.pallas import tpu_sc as plsc

NW = 32
C = 128
K = 2


def _lookup_call(tot, v, d, nch, ng, per_w):
    mesh = plsc.VectorSubcoreMesh(core_axis_name="c", subcore_axis_name="s")

    @functools.partial(
        pl.kernel,
        out_type=jax.ShapeDtypeStruct((tot, d), jnp.float32),
        mesh=mesh,
        scratch_types=[
            pltpu.VMEM_SHARED((v, d), jnp.float32),
            pltpu.VMEM((nch, C), jnp.int32),
            pltpu.VMEM((K * C, d), jnp.float32),
            pltpu.VMEM((K * C, d), jnp.float32),
            pltpu.SemaphoreType.DMA,
            pltpu.SemaphoreType.DMA,
            pltpu.SemaphoreType.DMA,
            pltpu.SemaphoreType.DMA,
        ],
    )
    def k(idx_hbm, table_hbm, out_hbm, table_sp, idx_all, b0, b1,
          gs0, gs1, ws0, ws1):
        bufs = (b0, b1)
        gsem = (gs0, gs1)
        wsem = (ws0, ws1)
        sid = lax.axis_index("s")
        wid = sid * 2 + lax.axis_index("c")
        base = wid * per_w

        @pl.when(sid == 0)
        def _():
            pltpu.sync_copy(table_hbm, table_sp)

        pltpu.sync_copy(idx_hbm.at[wid], idx_all)
        plsc.subcore_barrier()

        def gather(gr, h):
            return [pltpu.make_async_copy(
                table_sp.at[idx_all.at[gr * K + b]],
                bufs[h].at[pl.ds(b * C, C)], gsem[h])
                for b in range(K)]

        def write(gr, h):
            return [pltpu.make_async_copy(
                bufs[h], out_hbm.at[pl.ds(base + gr * K * C, K * C)],
                wsem[h])]

        for cp in gather(0, 0):
            cp.start()

        @pl.loop(0, ng, step=2)
        def _(g):
            for h in range(2):
                gr = g + h
                for cp in gather(gr, h):
                    cp.wait()
                for cp in write(gr, h):
                    cp.start()

                @pl.when(gr >= 1)
                def _():
                    for cp in write(gr - 1, 1 - h):
                        cp.wait()

                @pl.when(gr + 1 < ng)
                def _():
                    for cp in gather(gr + 1, 1 - h):
                        cp.start()

        for cp in write(ng - 1, 1):
            cp.wait()

    return k


def kernel(indices, weight):
    b, hist = indices.shape
    v, d = weight.shape
    tot = b * hist
    per_w = tot // NW
    nch = per_w // C
    ng = nch // K
    idx3 = indices.reshape(NW, nch, C)
    out = _lookup_call(tot, v, d, nch, ng, per_w)(idx3, weight)
    return out.reshape(b, hist, d)
